# baseline (speedup 1.0000x reference)
import functools

import jax
import jax.numpy as jnp
from jax import lax
from jax.experimental import pallas as pl
from jax.experimental.pallas import tpu as pltpu
from jax.experimental.pallas import tpu_sc as plsc

_N = 20000
_PAD = 20480
_NSUB = 16
_CHUNK = _PAD // _NSUB
_CVECS = _CHUNK // 16
_MAX_OUT = 300
_IOU_THRESH = 0.5
_NEG = -1e30


def _sc_body(b0, b1, b2, b3, s, out, x1v, y1v, x2v, y2v, arv, msv, headv,
             headsv, rowv, shared):
    sid = lax.axis_index("s")
    base = sid * _CHUNK
    pltpu.sync_copy(b0.at[pl.ds(base, _CHUNK)], x1v)
    pltpu.sync_copy(b2.at[pl.ds(base, _CHUNK)], x2v)
    pltpu.sync_copy(b1.at[pl.ds(base, _CHUNK)], y1v)
    pltpu.sync_copy(b3.at[pl.ds(base, _CHUNK)], y2v)
    pltpu.sync_copy(s.at[pl.ds(base, _CHUNK)], msv)
    lanes = lax.broadcasted_iota(jnp.int32, (16,), 0)

    def prep(j, _):
        sl = pl.ds(j * 16, 16)
        a = x1v[sl]
        b = x2v[sl]
        lo = jnp.minimum(a, b)
        hi = jnp.maximum(a, b)
        x1v[sl] = lo
        x2v[sl] = hi
        c = y1v[sl]
        d = y2v[sl]
        lo2 = jnp.minimum(c, d)
        hi2 = jnp.maximum(c, d)
        y1v[sl] = lo2
        y2v[sl] = hi2
        arv[sl] = (hi - lo) * (hi2 - lo2)
        return 0

    lax.fori_loop(0, _CVECS, prep, 0)

    def amax0(j, carry):
        bv, bi = carry
        v = msv[pl.ds(j * 16, 16)]
        li = j * 16 + lanes
        upd = v > bv
        return jnp.where(upd, v, bv), jnp.where(upd, li, bi)

    bv0, bi0 = lax.fori_loop(
        0, _CVECS, amax0,
        (jnp.full((16,), _NEG, jnp.float32), jnp.zeros((16,), jnp.int32)))

    def step(i, carry):
        bv, bi = carry
        m = jnp.max(bv)
        mi = jnp.min(jnp.where(bv == m, bi, jnp.int32(1 << 30)))
        gi = base + mi
        mi_v = jnp.zeros((16,), jnp.int32) + mi
        x1g = plsc.load_gather(x1v, [mi_v])
        y1g = plsc.load_gather(y1v, [mi_v])
        x2g = plsc.load_gather(x2v, [mi_v])
        y2g = plsc.load_gather(y2v, [mi_v])
        arg = plsc.load_gather(arv, [mi_v])
        headrow = jnp.where(
            lanes == 0, m,
            jnp.where(
                lanes == 1, gi.astype(jnp.float32),
                jnp.where(
                    lanes == 2, x1g,
                    jnp.where(
                        lanes == 3, y1g,
                        jnp.where(
                            lanes == 4, x2g,
                            jnp.where(lanes == 5, y2g,
                                      jnp.where(lanes == 6, arg, 0.0)))))))
        headv[...] = headrow
        pltpu.sync_copy(headv, shared.at[pl.ds(sid * 16, 16)])
        plsc.subcore_barrier()
        pltpu.sync_copy(shared, headsv)
        plsc.subcore_barrier()
        svec = plsc.load_gather(headsv, [lanes * 16])
        gm = jnp.max(svec)
        wk = jnp.min(jnp.where(svec == gm, lanes, jnp.int32(1 << 30)))
        valid = gm > jnp.float32(-5e29)
        wrow = plsc.load_gather(headsv, [wk * 16 + lanes])
        zf = jnp.float32(0.0)
        wgi = jnp.sum(jnp.where(lanes == 1, wrow, zf)).astype(jnp.int32)
        wx1 = jnp.sum(jnp.where(lanes == 2, wrow, zf))
        wy1 = jnp.sum(jnp.where(lanes == 3, wrow, zf))
        wx2 = jnp.sum(jnp.where(lanes == 4, wrow, zf))
        wy2 = jnp.sum(jnp.where(lanes == 5, wrow, zf))
        war = jnp.sum(jnp.where(lanes == 6, wrow, zf))

        def supp(j, nc):
            nbv, nbi = nc
            sl = pl.ds(j * 16, 16)
            msl = msv[sl]
            iw = jnp.maximum(
                jnp.minimum(x2v[sl], wx2) - jnp.maximum(x1v[sl], wx1), 0.0)
            ih = jnp.maximum(
                jnp.minimum(y2v[sl], wy2) - jnp.maximum(y1v[sl], wy1), 0.0)
            inter = iw * ih
            iou = inter / (arv[sl] + war - inter + jnp.float32(1e-8))
            lloc = j * 16 + lanes
            li = base + lloc
            kill = (iou > jnp.float32(_IOU_THRESH)) | (li == wgi)
            newms = jnp.where(kill, jnp.float32(_NEG), msl)
            msv[sl] = newms
            upd = newms > nbv
            return jnp.where(upd, newms, nbv), jnp.where(upd, lloc, nbi)

        nbv, nbi = lax.fori_loop(
            0, _CVECS, supp,
            (jnp.full((16,), _NEG, jnp.float32), jnp.zeros((16,), jnp.int32)))

        @pl.when(sid == 0)
        def _():
            vf = jnp.where(valid, jnp.float32(1.0), jnp.float32(0.0))
            rv = (jnp.where(lanes == 0, wx1, 0.0)
                  + jnp.where(lanes == 1, wy1, 0.0)
                  + jnp.where(lanes == 2, wx2, 0.0)
                  + jnp.where(lanes == 3, wy2, 0.0)
                  + jnp.where(lanes == 4, gm, 0.0)) * vf
            rowv[:] = rv
            pltpu.sync_copy(rowv, out.at[i])

        return nbv, nbi

    lax.fori_loop(0, _MAX_OUT, step, (bv0, bi0))


@functools.partial(jax.jit, static_argnames=())
def _sc_nms(b0, b1, b2, b3, sp):
    mesh = plsc.VectorSubcoreMesh(
        core_axis_name="c", subcore_axis_name="s", num_cores=1)
    run = pl.kernel(
        _sc_body,
        out_type=jax.ShapeDtypeStruct((_MAX_OUT, 16), jnp.float32),
        mesh=mesh,
        compiler_params=pltpu.CompilerParams(needs_layout_passes=False),
        scratch_types=[
            pltpu.VMEM((_CHUNK,), jnp.float32),
            pltpu.VMEM((_CHUNK,), jnp.float32),
            pltpu.VMEM((_CHUNK,), jnp.float32),
            pltpu.VMEM((_CHUNK,), jnp.float32),
            pltpu.VMEM((_CHUNK,), jnp.float32),
            pltpu.VMEM((_CHUNK,), jnp.float32),
            pltpu.VMEM((16,), jnp.float32),
            pltpu.VMEM((_NSUB * 16,), jnp.float32),
            pltpu.VMEM((16,), jnp.float32),
            pltpu.VMEM_SHARED((_NSUB * 16,), jnp.float32),
        ],
    )
    return run(b0, b1, b2, b3, sp)


def kernel(boxes, scores):
    bp = jnp.pad(boxes, ((0, _PAD - _N), (0, 0)))
    sp = jnp.pad(scores, (0, _PAD - _N), constant_values=-1e30)
    out = _sc_nms(bp[:, 0], bp[:, 1], bp[:, 2], bp[:, 3], sp)
    return out[:, :5]


# speedup vs baseline: 1.7553x; 1.7553x over previous
import functools

import jax
import jax.numpy as jnp
from jax import lax
from jax.experimental import pallas as pl
from jax.experimental.pallas import tpu as pltpu
from jax.experimental.pallas import tpu_sc as plsc

_N = 20000
_PAD = 20480
_NSUB = 16
_CHUNK = _PAD // _NSUB
_CVECS = _CHUNK // 16
_MAX_OUT = 300
_IOU_THRESH = 0.5
_NEG = -1e30


def _sc_body(b0, b1, b2, b3, s, out, x1v, y1v, x2v, y2v, arv, msv, headv,
             headsv, rowv, shared):
    sid = lax.axis_index("s")
    base = sid * _CHUNK
    pltpu.sync_copy(b0.at[pl.ds(base, _CHUNK)], x1v)
    pltpu.sync_copy(b2.at[pl.ds(base, _CHUNK)], x2v)
    pltpu.sync_copy(b1.at[pl.ds(base, _CHUNK)], y1v)
    pltpu.sync_copy(b3.at[pl.ds(base, _CHUNK)], y2v)
    pltpu.sync_copy(s.at[pl.ds(base, _CHUNK)], msv)
    lanes = lax.broadcasted_iota(jnp.int32, (16,), 0)

    def prep(j, _):
        sl = pl.ds(j * 16, 16)
        a = x1v[sl]
        b = x2v[sl]
        lo = jnp.minimum(a, b)
        hi = jnp.maximum(a, b)
        x1v[sl] = lo
        x2v[sl] = hi
        c = y1v[sl]
        d = y2v[sl]
        lo2 = jnp.minimum(c, d)
        hi2 = jnp.maximum(c, d)
        y1v[sl] = lo2
        y2v[sl] = hi2
        arv[sl] = (hi - lo) * (hi2 - lo2)
        return 0

    lax.fori_loop(0, _CVECS, prep, 0)

    def step(i, _):
        def amax(j, carry):
            bv, bi = carry
            v = msv[pl.ds(j * 16, 16)]
            li = j * 16 + lanes
            upd = v > bv
            return jnp.where(upd, v, bv), jnp.where(upd, li, bi)

        bv, bi = lax.fori_loop(
            0, _CVECS, amax,
            (jnp.full((16,), _NEG, jnp.float32), jnp.zeros((16,), jnp.int32)))
        m = jnp.max(bv)
        mi = jnp.min(jnp.where(bv == m, bi, jnp.int32(1 << 30)))
        gi = base + mi
        mi_v = jnp.zeros((16,), jnp.int32) + mi
        x1g = plsc.load_gather(x1v, [mi_v])
        y1g = plsc.load_gather(y1v, [mi_v])
        x2g = plsc.load_gather(x2v, [mi_v])
        y2g = plsc.load_gather(y2v, [mi_v])
        arg = plsc.load_gather(arv, [mi_v])
        headrow = jnp.where(
            lanes == 0, m,
            jnp.where(
                lanes == 1, gi.astype(jnp.float32),
                jnp.where(
                    lanes == 2, x1g,
                    jnp.where(
                        lanes == 3, y1g,
                        jnp.where(
                            lanes == 4, x2g,
                            jnp.where(lanes == 5, y2g,
                                      jnp.where(lanes == 6, arg, 0.0)))))))
        headv[...] = headrow
        pltpu.sync_copy(headv, shared.at[pl.ds(sid * 16, 16)])
        plsc.subcore_barrier()
        pltpu.sync_copy(shared, headsv)
        plsc.subcore_barrier()
        svec = plsc.load_gather(headsv, [lanes * 16])
        gm = jnp.max(svec)
        wk = jnp.min(jnp.where(svec == gm, lanes, jnp.int32(1 << 30)))
        valid = gm > jnp.float32(-5e29)
        wrow = plsc.load_gather(headsv, [wk * 16 + lanes])
        zf = jnp.float32(0.0)
        wgi = jnp.sum(jnp.where(lanes == 1, wrow, zf)).astype(jnp.int32)
        wx1 = jnp.sum(jnp.where(lanes == 2, wrow, zf))
        wy1 = jnp.sum(jnp.where(lanes == 3, wrow, zf))
        wx2 = jnp.sum(jnp.where(lanes == 4, wrow, zf))
        wy2 = jnp.sum(jnp.where(lanes == 5, wrow, zf))
        war = jnp.sum(jnp.where(lanes == 6, wrow, zf))

        def supp(j, _):
            sl = pl.ds(j * 16, 16)
            msl = msv[sl]
            iw = jnp.maximum(
                jnp.minimum(x2v[sl], wx2) - jnp.maximum(x1v[sl], wx1), 0.0)
            ih = jnp.maximum(
                jnp.minimum(y2v[sl], wy2) - jnp.maximum(y1v[sl], wy1), 0.0)
            inter = iw * ih
            iou = inter / (arv[sl] + war - inter + jnp.float32(1e-8))
            li = base + j * 16 + lanes
            kill = (iou > jnp.float32(_IOU_THRESH)) | (li == wgi)
            msv[sl] = jnp.where(kill, jnp.float32(_NEG), msl)
            return 0

        lax.fori_loop(0, _CVECS, supp, 0)

        @pl.when(sid == 0)
        def _():
            vf = jnp.where(valid, jnp.float32(1.0), jnp.float32(0.0))
            rv = (jnp.where(lanes == 0, wx1, 0.0)
                  + jnp.where(lanes == 1, wy1, 0.0)
                  + jnp.where(lanes == 2, wx2, 0.0)
                  + jnp.where(lanes == 3, wy2, 0.0)
                  + jnp.where(lanes == 4, gm, 0.0)) * vf
            rowv[:] = rv
            pltpu.sync_copy(rowv, out.at[i])

        return 0

    lax.fori_loop(0, _MAX_OUT, step, 0)


@functools.partial(jax.jit, static_argnames=())
def _sc_nms(b0, b1, b2, b3, sp):
    mesh = plsc.VectorSubcoreMesh(
        core_axis_name="c", subcore_axis_name="s", num_cores=1)
    run = pl.kernel(
        _sc_body,
        out_type=jax.ShapeDtypeStruct((_MAX_OUT, 16), jnp.float32),
        mesh=mesh,
        compiler_params=pltpu.CompilerParams(needs_layout_passes=False),
        scratch_types=[
            pltpu.VMEM((_CHUNK,), jnp.float32),
            pltpu.VMEM((_CHUNK,), jnp.float32),
            pltpu.VMEM((_CHUNK,), jnp.float32),
            pltpu.VMEM((_CHUNK,), jnp.float32),
            pltpu.VMEM((_CHUNK,), jnp.float32),
            pltpu.VMEM((_CHUNK,), jnp.float32),
            pltpu.VMEM((16,), jnp.float32),
            pltpu.VMEM((_NSUB * 16,), jnp.float32),
            pltpu.VMEM((16,), jnp.float32),
            pltpu.VMEM_SHARED((_NSUB * 16,), jnp.float32),
        ],
    )
    return run(b0, b1, b2, b3, sp)


def kernel(boxes, scores):
    bp = jnp.pad(boxes, ((0, _PAD - _N), (0, 0)))
    sp = jnp.pad(scores, (0, _PAD - _N), constant_values=-1e30)
    out = _sc_nms(bp[:, 0], bp[:, 1], bp[:, 2], bp[:, 3], sp)
    return out[:, :5]


# speedup vs baseline: 2.3660x; 1.3479x over previous
import functools

import jax
import jax.numpy as jnp
from jax import lax
from jax.experimental import pallas as pl
from jax.experimental.pallas import tpu as pltpu
from jax.experimental.pallas import tpu_sc as plsc

_N = 20000
_PAD = 20480
_NSUB = 16
_CHUNK = _PAD // _NSUB
_CVECS = _CHUNK // 16
_MAX_OUT = 300
_IOU_THRESH = 0.5
_NEG = -1e30


def _sc_body(b0, b1, b2, b3, s, out, x1v, y1v, x2v, y2v, arv, msv, headv,
             headsv, rowv, shared):
    sid = lax.axis_index("s")
    base = sid * _CHUNK
    pltpu.sync_copy(b0.at[pl.ds(base, _CHUNK)], x1v)
    pltpu.sync_copy(b2.at[pl.ds(base, _CHUNK)], x2v)
    pltpu.sync_copy(b1.at[pl.ds(base, _CHUNK)], y1v)
    pltpu.sync_copy(b3.at[pl.ds(base, _CHUNK)], y2v)
    pltpu.sync_copy(s.at[pl.ds(base, _CHUNK)], msv)
    lanes = lax.broadcasted_iota(jnp.int32, (16,), 0)

    def prep(j, _):
        sl = pl.ds(j * 16, 16)
        a = x1v[sl]
        b = x2v[sl]
        lo = jnp.minimum(a, b)
        hi = jnp.maximum(a, b)
        x1v[sl] = lo
        x2v[sl] = hi
        c = y1v[sl]
        d = y2v[sl]
        lo2 = jnp.minimum(c, d)
        hi2 = jnp.maximum(c, d)
        y1v[sl] = lo2
        y2v[sl] = hi2
        arv[sl] = (hi - lo) * (hi2 - lo2)
        return 0

    lax.fori_loop(0, _CVECS, prep, 0)

    def step(i, _):
        def amax(j, carry):
            out = []
            for q in range(4):
                bv, bi = carry[q]
                jj = j + q * (_CVECS // 4)
                v = msv[pl.ds(jj * 16, 16)]
                li = jj * 16 + lanes
                upd = v > bv
                out.append((jnp.where(upd, v, bv), jnp.where(upd, li, bi)))
            return tuple(out)

        init = tuple(
            (jnp.full((16,), _NEG, jnp.float32), jnp.zeros((16,), jnp.int32))
            for _ in range(4))
        quads = lax.fori_loop(0, _CVECS // 4, amax, init)
        bv, bi = quads[0]
        for q in range(1, 4):
            qv, qi = quads[q]
            take = (qv > bv) | ((qv == bv) & (qi < bi))
            bv = jnp.where(take, qv, bv)
            bi = jnp.where(take, qi, bi)
        m = jnp.max(bv)
        mi = jnp.min(jnp.where(bv == m, bi, jnp.int32(1 << 30)))
        gi = base + mi
        mi_v = jnp.zeros((16,), jnp.int32) + mi
        x1g = plsc.load_gather(x1v, [mi_v])
        y1g = plsc.load_gather(y1v, [mi_v])
        x2g = plsc.load_gather(x2v, [mi_v])
        y2g = plsc.load_gather(y2v, [mi_v])
        arg = plsc.load_gather(arv, [mi_v])
        headrow = jnp.where(
            lanes == 0, m,
            jnp.where(
                lanes == 1, gi.astype(jnp.float32),
                jnp.where(
                    lanes == 2, x1g,
                    jnp.where(
                        lanes == 3, y1g,
                        jnp.where(
                            lanes == 4, x2g,
                            jnp.where(lanes == 5, y2g,
                                      jnp.where(lanes == 6, arg, 0.0)))))))
        headv[...] = headrow
        pltpu.sync_copy(headv, shared.at[pl.ds(sid * 16, 16)])
        plsc.subcore_barrier()
        pltpu.sync_copy(shared, headsv)
        plsc.subcore_barrier()
        svec = plsc.load_gather(headsv, [lanes * 16])
        gm = jnp.max(svec)
        wk = jnp.min(jnp.where(svec == gm, lanes, jnp.int32(1 << 30)))
        valid = gm > jnp.float32(-5e29)
        wrow = plsc.load_gather(headsv, [wk * 16 + lanes])
        zf = jnp.float32(0.0)
        wgi = jnp.sum(jnp.where(lanes == 1, wrow, zf)).astype(jnp.int32)
        wx1 = jnp.sum(jnp.where(lanes == 2, wrow, zf))
        wy1 = jnp.sum(jnp.where(lanes == 3, wrow, zf))
        wx2 = jnp.sum(jnp.where(lanes == 4, wrow, zf))
        wy2 = jnp.sum(jnp.where(lanes == 5, wrow, zf))
        war = jnp.sum(jnp.where(lanes == 6, wrow, zf))

        def supp(j, _):
            for q in range(4):
                jj = j * 4 + q
                sl = pl.ds(jj * 16, 16)
                msl = msv[sl]
                iw = jnp.maximum(
                    jnp.minimum(x2v[sl], wx2) - jnp.maximum(x1v[sl], wx1), 0.0)
                ih = jnp.maximum(
                    jnp.minimum(y2v[sl], wy2) - jnp.maximum(y1v[sl], wy1), 0.0)
                inter = iw * ih
                iou = inter / (arv[sl] + war - inter + jnp.float32(1e-8))
                li = base + jj * 16 + lanes
                kill = (iou > jnp.float32(_IOU_THRESH)) | (li == wgi)
                msv[sl] = jnp.where(kill, jnp.float32(_NEG), msl)
            return 0

        lax.fori_loop(0, _CVECS // 4, supp, 0)

        @pl.when(sid == 0)
        def _():
            vf = jnp.where(valid, jnp.float32(1.0), jnp.float32(0.0))
            rv = (jnp.where(lanes == 0, wx1, 0.0)
                  + jnp.where(lanes == 1, wy1, 0.0)
                  + jnp.where(lanes == 2, wx2, 0.0)
                  + jnp.where(lanes == 3, wy2, 0.0)
                  + jnp.where(lanes == 4, gm, 0.0)) * vf
            rowv[:] = rv
            pltpu.sync_copy(rowv, out.at[i])

        return 0

    lax.fori_loop(0, _MAX_OUT, step, 0)


@functools.partial(jax.jit, static_argnames=())
def _sc_nms(b0, b1, b2, b3, sp):
    mesh = plsc.VectorSubcoreMesh(
        core_axis_name="c", subcore_axis_name="s", num_cores=1)
    run = pl.kernel(
        _sc_body,
        out_type=jax.ShapeDtypeStruct((_MAX_OUT, 16), jnp.float32),
        mesh=mesh,
        compiler_params=pltpu.CompilerParams(needs_layout_passes=False),
        scratch_types=[
            pltpu.VMEM((_CHUNK,), jnp.float32),
            pltpu.VMEM((_CHUNK,), jnp.float32),
            pltpu.VMEM((_CHUNK,), jnp.float32),
            pltpu.VMEM((_CHUNK,), jnp.float32),
            pltpu.VMEM((_CHUNK,), jnp.float32),
            pltpu.VMEM((_CHUNK,), jnp.float32),
            pltpu.VMEM((16,), jnp.float32),
            pltpu.VMEM((_NSUB * 16,), jnp.float32),
            pltpu.VMEM((16,), jnp.float32),
            pltpu.VMEM_SHARED((_NSUB * 16,), jnp.float32),
        ],
    )
    return run(b0, b1, b2, b3, sp)


def kernel(boxes, scores):
    bp = jnp.pad(boxes, ((0, _PAD - _N), (0, 0)))
    sp = jnp.pad(scores, (0, _PAD - _N), constant_values=-1e30)
    out = _sc_nms(bp[:, 0], bp[:, 1], bp[:, 2], bp[:, 3], sp)
    return out[:, :5]
